# baseline (speedup 1.0000x reference)
import functools

import jax
import jax.numpy as jnp
from jax import lax
from jax.experimental import pallas as pl
from jax.experimental.pallas import tpu as pltpu
from jax.experimental.pallas import tpu_sc as plsc

N = 10000
E = 160000
D = 256
NPAD = 10240
NC = 2
NSUB = 16
NW = NC * NSUB
NR = NPAD // NW
SCH = 2000
NSC = E // SCH
GR = 128
DEGW = 16

_f32 = jnp.float32


def _prep_body(dst_hbm, deg_hbm, dst_v, floc_v, acc_v):
    c = lax.axis_index("c")
    s = lax.axis_index("s")
    w = s * NC + c
    base = w * NR
    z16 = jnp.zeros((16,), _f32)
    zi16 = jnp.zeros((16,), jnp.int32)
    one16 = jnp.ones((16,), _f32)

    def zacc(i, _):
        acc_v[pl.ds(i * 16, 16)] = z16
        return 0

    lax.fori_loop(0, NR * DEGW // 16, zacc, 0)

    for i in range((SCH + 16) // 16):
        floc_v[pl.ds(i * 16, 16)] = zi16

    def superchunk(t, _):
        pltpu.sync_copy(dst_hbm.at[pl.ds(t * SCH, SCH)], dst_v)

        def scan(g, pos):
            dstv = dst_v[pl.ds(g * 16, 16)]
            loc = dstv - base
            ok = (loc >= 0) & (loc < NR)
            locc = jnp.where(ok, loc, 0)
            idx = pos + plsc.cumsum(jnp.ones((16,), jnp.int32), mask=ok) - 1
            plsc.store_scatter(floc_v, [idx], locc, mask=ok)
            return pos + plsc.all_reduce_population_count(ok)[0]

        cnt = lax.fori_loop(0, SCH // 16, scan, jnp.int32(0))

        def edge(e, _):
            loc = floc_v[pl.ds(e, 16)][0]
            plsc.addupdate(acc_v.at[pl.ds(loc * DEGW, 16)], one16)
            return 0

        lax.fori_loop(0, cnt, edge, 0)
        return 0

    lax.fori_loop(0, NSC, superchunk, 0)
    pltpu.sync_copy(acc_v, deg_hbm.at[pl.ds(base * DEGW, NR * DEGW)])


_sc_prep = functools.partial(
    pl.kernel,
    out_type=jax.ShapeDtypeStruct((NPAD * DEGW,), _f32),
    mesh=plsc.VectorSubcoreMesh(core_axis_name="c", subcore_axis_name="s"),
    scratch_types=[
        pltpu.VMEM((SCH,), jnp.int32),
        pltpu.VMEM((SCH + 16,), jnp.int32),
        pltpu.VMEM((NR * DEGW,), _f32),
    ],
    compiler_params=pltpu.CompilerParams(needs_layout_passes=False),
)(_prep_body)


def _agg_body(p_hbm, src_hbm, dst_hbm, s_hbm, src_v, dst_v, fsrc_v, floc_v,
              buf_v, acc_v, sem):
    c = lax.axis_index("c")
    s = lax.axis_index("s")
    w = s * NC + c
    base = w * NR
    z16 = jnp.zeros((16,), _f32)
    zi16 = jnp.zeros((16,), jnp.int32)

    def zacc(i, _):
        acc_v[pl.ds(i * 16, 16)] = z16
        return 0

    lax.fori_loop(0, NR * D // 16, zacc, 0)

    for i in range((SCH + 16) // 16):
        fsrc_v[pl.ds(i * 16, 16)] = zi16
        floc_v[pl.ds(i * 16, 16)] = zi16

    def superchunk(t, _):
        pltpu.sync_copy(src_hbm.at[pl.ds(t * SCH, SCH)], src_v)
        pltpu.sync_copy(dst_hbm.at[pl.ds(t * SCH, SCH)], dst_v)

        def scan(g, pos):
            dstv = dst_v[pl.ds(g * 16, 16)]
            loc = dstv - base
            ok = (loc >= 0) & (loc < NR)
            srcv = src_v[pl.ds(g * 16, 16)]
            locc = jnp.where(ok, loc, 0)
            idx = pos + plsc.cumsum(jnp.ones((16,), jnp.int32), mask=ok) - 1
            plsc.store_scatter(fsrc_v, [idx], srcv, mask=ok)
            plsc.store_scatter(floc_v, [idx], locc, mask=ok)
            return pos + plsc.all_reduce_population_count(ok)[0]

        cnt = lax.fori_loop(0, SCH // 16, scan, jnp.int32(0))
        nr_rounds = (cnt + GR - 1) // GR

        def rnd(r, _):
            pltpu.async_copy(
                p_hbm.at[fsrc_v.at[pl.ds(r * GR, GR)]], buf_v, sem
            ).wait()
            e_hi = jnp.minimum(GR, cnt - r * GR)

            def edge(e, _):
                loc = floc_v[pl.ds(r * GR + e, 16)][0]
                for g in range(D // 16):
                    xv = buf_v[e, pl.ds(g * 16, 16)]
                    plsc.addupdate(acc_v.at[pl.ds(loc * D + g * 16, 16)], xv)
                return 0

            lax.fori_loop(0, e_hi, edge, 0)
            return 0

        lax.fori_loop(0, nr_rounds, rnd, 0)
        return 0

    lax.fori_loop(0, NSC, superchunk, 0)
    pltpu.sync_copy(acc_v, s_hbm.at[pl.ds(base * D, NR * D)])


_sc_agg = functools.partial(
    pl.kernel,
    out_type=jax.ShapeDtypeStruct((NPAD * D,), _f32),
    mesh=plsc.VectorSubcoreMesh(core_axis_name="c", subcore_axis_name="s"),
    scratch_types=[
        pltpu.VMEM((SCH,), jnp.int32),
        pltpu.VMEM((SCH,), jnp.int32),
        pltpu.VMEM((SCH + 16,), jnp.int32),
        pltpu.VMEM((SCH + 16,), jnp.int32),
        pltpu.VMEM((GR, D), _f32),
        pltpu.VMEM((NR * D,), _f32),
        pltpu.SemaphoreType.DMA,
    ],
    compiler_params=pltpu.CompilerParams(needs_layout_passes=False),
)(_agg_body)


def _tc1_body(x_ref, deg_ref, wpre_ref, bpre_ref, wmp1_ref, out_ref):
    h = jnp.dot(x_ref[...], wpre_ref[...], preferred_element_type=_f32)
    h = jnp.maximum(h + bpre_ref[...], 0.0)
    p = jnp.dot(h, wmp1_ref[...], preferred_element_type=_f32)
    dinv = lax.rsqrt(deg_ref[...] + 1.0)
    out_ref[...] = p * dinv


def _tc2_body(s_ref, p_ref, deg_ref, b_ref, w_ref, out_ref):
    dinv = lax.rsqrt(deg_ref[...] + 1.0)
    h = jnp.maximum(dinv * (s_ref[...] + p_ref[...]) + b_ref[...], 0.0)
    out_ref[...] = jnp.dot(h, w_ref[...], preferred_element_type=_f32) * dinv


def _tc3_body(s_ref, p_ref, deg_ref, b2_ref, wpo_ref, bpo_ref, wfc_ref, bfc_ref,
              out_ref):
    dinv = lax.rsqrt(deg_ref[...] + 1.0)
    h = jnp.maximum(dinv * (s_ref[...] + p_ref[...]) + b2_ref[...], 0.0)
    h = jnp.dot(h, wpo_ref[...], preferred_element_type=_f32)
    h = jnp.maximum(h + bpo_ref[...], 0.0)
    out_ref[...] = jnp.dot(h, wfc_ref[...], preferred_element_type=_f32) + bfc_ref[...]


_BLK = 256
_GRID = NPAD // _BLK


def _row_spec():
    return pl.BlockSpec((_BLK, D), lambda i: (i, 0))


def _deg_spec():
    return pl.BlockSpec((_BLK, 1), lambda i: (i, 0))


def _w_spec():
    return pl.BlockSpec((D, D), lambda i: (0, 0))


def _b_spec():
    return pl.BlockSpec((1, D), lambda i: (0, 0))


_tc1 = pl.pallas_call(
    _tc1_body,
    grid=(_GRID,),
    in_specs=[_row_spec(), _deg_spec(), _w_spec(), _b_spec(), _w_spec()],
    out_specs=_row_spec(),
    out_shape=jax.ShapeDtypeStruct((NPAD, D), _f32),
)

_tc2 = pl.pallas_call(
    _tc2_body,
    grid=(_GRID,),
    in_specs=[_row_spec(), _row_spec(), _deg_spec(), _b_spec(), _w_spec()],
    out_specs=_row_spec(),
    out_shape=jax.ShapeDtypeStruct((NPAD, D), _f32),
)

_tc3 = pl.pallas_call(
    _tc3_body,
    grid=(_GRID,),
    in_specs=[_row_spec(), _row_spec(), _deg_spec(), _b_spec(), _w_spec(),
              _b_spec(), _w_spec(), _b_spec()],
    out_specs=_row_spec(),
    out_shape=jax.ShapeDtypeStruct((NPAD, D), _f32),
)


def kernel(x, edge_index, W_pre, b_pre, W_mp1, b_mp1, W_mp2, b_mp2, W_post,
           b_post, W_fc, b_fc):
    src = edge_index[0]
    dst = edge_index[1]
    deg = _sc_prep(dst).reshape(NPAD, DEGW)[:, :1]
    xp = jnp.pad(x, ((0, NPAD - N), (0, 0)))
    p1 = _tc1(xp, deg, W_pre, b_pre.reshape(1, D), W_mp1)
    s1 = _sc_agg(p1, src, dst).reshape(NPAD, D)
    p2 = _tc2(s1, p1, deg, b_mp1.reshape(1, D), W_mp2)
    s2 = _sc_agg(p2, src, dst).reshape(NPAD, D)
    out = _tc3(s2, p2, deg, b_mp2.reshape(1, D), W_post, b_post.reshape(1, D),
               W_fc, b_fc.reshape(1, D))
    return out[:N]


# speedup vs baseline: 4.4231x; 4.4231x over previous
import functools

import jax
import jax.numpy as jnp
from jax import lax
from jax.experimental import pallas as pl
from jax.experimental.pallas import tpu as pltpu
from jax.experimental.pallas import tpu_sc as plsc

N = 10000
E = 160000
D = 256
NPAD = 10240
NC = 2
NSUB = 16
NW = NC * NSUB
NR = NPAD // NW
SCH = 2000
NSC = E // SCH
GR = 32
DEGW = 16

_f32 = jnp.float32


def _prep_body(dst_hbm, deg_hbm, dst_v, floc_v, acc_v):
    c = lax.axis_index("c")
    s = lax.axis_index("s")
    w = s * NC + c
    base = w * NR
    z16 = jnp.zeros((16,), _f32)
    zi16 = jnp.zeros((16,), jnp.int32)
    one16 = jnp.ones((16,), _f32)

    def zacc(i, _):
        acc_v[pl.ds(i * 16, 16)] = z16
        return 0

    lax.fori_loop(0, NR * DEGW // 16, zacc, 0)

    for i in range((SCH + 16) // 16):
        floc_v[pl.ds(i * 16, 16)] = zi16

    def superchunk(t, _):
        pltpu.sync_copy(dst_hbm.at[pl.ds(t * SCH, SCH)], dst_v)

        def scan(g, pos):
            dstv = dst_v[pl.ds(g * 16, 16)]
            loc = dstv - base
            ok = (loc >= 0) & (loc < NR)
            locc = jnp.where(ok, loc, 0)
            idx = pos + plsc.cumsum(jnp.ones((16,), jnp.int32), mask=ok) - 1
            plsc.store_scatter(floc_v, [idx], locc, mask=ok)
            return pos + plsc.all_reduce_population_count(ok)[0]

        cnt = lax.fori_loop(0, SCH // 16, scan, jnp.int32(0))

        def edge(e, _):
            loc = floc_v[pl.ds(e, 16)][0]
            plsc.addupdate(acc_v.at[pl.ds(loc * DEGW, 16)], one16)
            return 0

        lax.fori_loop(0, cnt, edge, 0)
        return 0

    lax.fori_loop(0, NSC, superchunk, 0)
    pltpu.sync_copy(acc_v, deg_hbm.at[pl.ds(base * DEGW, NR * DEGW)])


_sc_prep = functools.partial(
    pl.kernel,
    out_type=jax.ShapeDtypeStruct((NPAD * DEGW,), _f32),
    mesh=plsc.VectorSubcoreMesh(core_axis_name="c", subcore_axis_name="s"),
    scratch_types=[
        pltpu.VMEM((SCH,), jnp.int32),
        pltpu.VMEM((SCH + 16,), jnp.int32),
        pltpu.VMEM((NR * DEGW,), _f32),
    ],
    compiler_params=pltpu.CompilerParams(needs_layout_passes=False),
)(_prep_body)


def _agg_body(p_hbm, src_hbm, dst_hbm, s_hbm, src_v, dst_v, fsrc_v, floc_v,
              buf_v, acc_v, sem):
    c = lax.axis_index("c")
    s = lax.axis_index("s")
    w = s * NC + c
    base = w * NR
    z16 = jnp.zeros((16,), _f32)
    zi16 = jnp.zeros((16,), jnp.int32)

    def zacc(i, _):
        acc_v[pl.ds(i * 16, 16)] = z16
        return 0

    lax.fori_loop(0, NR * D // 16, zacc, 0)

    for i in range((SCH + 16) // 16):
        fsrc_v[pl.ds(i * 16, 16)] = zi16
        floc_v[pl.ds(i * 16, 16)] = zi16

    def superchunk(t, _):
        pltpu.sync_copy(src_hbm.at[pl.ds(t * SCH, SCH)], src_v)
        pltpu.sync_copy(dst_hbm.at[pl.ds(t * SCH, SCH)], dst_v)

        def scan(g, pos):
            dstv = dst_v[pl.ds(g * 16, 16)]
            loc = dstv - base
            ok = (loc >= 0) & (loc < NR)
            srcv = src_v[pl.ds(g * 16, 16)]
            locc = jnp.where(ok, loc, 0)
            idx = pos + plsc.cumsum(jnp.ones((16,), jnp.int32), mask=ok) - 1
            plsc.store_scatter(fsrc_v, [idx], srcv, mask=ok)
            plsc.store_scatter(floc_v, [idx], locc, mask=ok)
            return pos + plsc.all_reduce_population_count(ok)[0]

        cnt = lax.fori_loop(0, SCH // 16, scan, jnp.int32(0))
        nr_rounds = (cnt + GR - 1) // GR

        def rnd(r, _):
            pltpu.async_copy(
                p_hbm.at[fsrc_v.at[pl.ds(r * GR, GR)]], buf_v, sem
            ).wait()
            e_hi = jnp.minimum(GR, cnt - r * GR)

            def edge(e, _):
                loc = floc_v[pl.ds(r * GR + e, 16)][0]
                for g in range(D // 16):
                    xv = buf_v[e, pl.ds(g * 16, 16)]
                    plsc.addupdate(acc_v.at[pl.ds(loc * D + g * 16, 16)], xv)
                return 0

            lax.fori_loop(0, e_hi, edge, 0)
            return 0

        lax.fori_loop(0, nr_rounds, rnd, 0)
        return 0

    lax.fori_loop(0, NSC, superchunk, 0)
    pltpu.sync_copy(acc_v, s_hbm.at[pl.ds(base * D, NR * D)])


_sc_agg = functools.partial(
    pl.kernel,
    out_type=jax.ShapeDtypeStruct((NPAD * D,), _f32),
    mesh=plsc.VectorSubcoreMesh(core_axis_name="c", subcore_axis_name="s"),
    scratch_types=[
        pltpu.VMEM((SCH,), jnp.int32),
        pltpu.VMEM((SCH,), jnp.int32),
        pltpu.VMEM((SCH + 16,), jnp.int32),
        pltpu.VMEM((SCH + 16,), jnp.int32),
        pltpu.VMEM((GR, D), _f32),
        pltpu.VMEM((NR * D,), _f32),
        pltpu.SemaphoreType.DMA,
    ],
    compiler_params=pltpu.CompilerParams(needs_layout_passes=False),
)(_agg_body)


def _tc1_body(x_ref, deg_ref, wpre_ref, bpre_ref, wmp1_ref, out_ref):
    h = jnp.dot(x_ref[...], wpre_ref[...], preferred_element_type=_f32)
    h = jnp.maximum(h + bpre_ref[...], 0.0)
    p = jnp.dot(h, wmp1_ref[...], preferred_element_type=_f32)
    dinv = lax.rsqrt(deg_ref[...] + 1.0)
    out_ref[...] = p * dinv


def _tc2_body(s_ref, p_ref, deg_ref, b_ref, w_ref, out_ref):
    dinv = lax.rsqrt(deg_ref[...] + 1.0)
    h = jnp.maximum(dinv * (s_ref[...] + p_ref[...]) + b_ref[...], 0.0)
    out_ref[...] = jnp.dot(h, w_ref[...], preferred_element_type=_f32) * dinv


def _tc3_body(s_ref, p_ref, deg_ref, b2_ref, wpo_ref, bpo_ref, wfc_ref, bfc_ref,
              out_ref):
    dinv = lax.rsqrt(deg_ref[...] + 1.0)
    h = jnp.maximum(dinv * (s_ref[...] + p_ref[...]) + b2_ref[...], 0.0)
    h = jnp.dot(h, wpo_ref[...], preferred_element_type=_f32)
    h = jnp.maximum(h + bpo_ref[...], 0.0)
    out_ref[...] = jnp.dot(h, wfc_ref[...], preferred_element_type=_f32) + bfc_ref[...]


_BLK = 256
_GRID = NPAD // _BLK


def _row_spec():
    return pl.BlockSpec((_BLK, D), lambda i: (i, 0))


def _deg_spec():
    return pl.BlockSpec((_BLK, 1), lambda i: (i, 0))


def _w_spec():
    return pl.BlockSpec((D, D), lambda i: (0, 0))


def _b_spec():
    return pl.BlockSpec((1, D), lambda i: (0, 0))


_tc1 = pl.pallas_call(
    _tc1_body,
    grid=(_GRID,),
    in_specs=[_row_spec(), _deg_spec(), _w_spec(), _b_spec(), _w_spec()],
    out_specs=_row_spec(),
    out_shape=jax.ShapeDtypeStruct((NPAD, D), _f32),
)

_tc2 = pl.pallas_call(
    _tc2_body,
    grid=(_GRID,),
    in_specs=[_row_spec(), _row_spec(), _deg_spec(), _b_spec(), _w_spec()],
    out_specs=_row_spec(),
    out_shape=jax.ShapeDtypeStruct((NPAD, D), _f32),
)

_tc3 = pl.pallas_call(
    _tc3_body,
    grid=(_GRID,),
    in_specs=[_row_spec(), _row_spec(), _deg_spec(), _b_spec(), _w_spec(),
              _b_spec(), _w_spec(), _b_spec()],
    out_specs=_row_spec(),
    out_shape=jax.ShapeDtypeStruct((NPAD, D), _f32),
)


def kernel(x, edge_index, W_pre, b_pre, W_mp1, b_mp1, W_mp2, b_mp2, W_post,
           b_post, W_fc, b_fc):
    src = edge_index[0]
    dst = edge_index[1]
    deg = _sc_prep(dst).reshape(NPAD, DEGW)[:, :1]
    xp = jnp.pad(x, ((0, NPAD - N), (0, 0)))
    p1 = _tc1(xp, deg, W_pre, b_pre.reshape(1, D), W_mp1)
    s1 = _sc_agg(p1, src, dst).reshape(NPAD, D)
    p2 = _tc2(s1, p1, deg, b_mp1.reshape(1, D), W_mp2)
    s2 = _sc_agg(p2, src, dst).reshape(NPAD, D)
    out = _tc3(s2, p2, deg, b_mp2.reshape(1, D), W_post, b_post.reshape(1, D),
               W_fc, b_fc.reshape(1, D))
    return out[:N]


# speedup vs baseline: 4.7144x; 1.0659x over previous
import functools

import jax
import jax.numpy as jnp
from jax import lax
from jax.experimental import pallas as pl
from jax.experimental.pallas import tpu as pltpu
from jax.experimental.pallas import tpu_sc as plsc

N = 10000
E = 160000
D = 256
NPAD = 10240
NC = 2
NSUB = 16
NW = NC * NSUB
NR = NPAD // NW
SCH = 2000
NSC = E // SCH
GR = 16
DEGW = 16

_f32 = jnp.float32


def _prep_body(dst_hbm, deg_hbm, dst_v, floc_v, acc_v):
    c = lax.axis_index("c")
    s = lax.axis_index("s")
    w = s * NC + c
    base = w * NR
    z16 = jnp.zeros((16,), _f32)
    zi16 = jnp.zeros((16,), jnp.int32)
    one16 = jnp.ones((16,), _f32)

    def zacc(i, _):
        acc_v[pl.ds(i * 16, 16)] = z16
        return 0

    lax.fori_loop(0, NR * DEGW // 16, zacc, 0)

    for i in range((SCH + 16) // 16):
        floc_v[pl.ds(i * 16, 16)] = zi16

    def superchunk(t, _):
        pltpu.sync_copy(dst_hbm.at[pl.ds(t * SCH, SCH)], dst_v)

        def scan(g, pos):
            dstv = dst_v[pl.ds(g * 16, 16)]
            loc = dstv - base
            ok = (loc >= 0) & (loc < NR)
            locc = jnp.where(ok, loc, 0)
            idx = pos + plsc.cumsum(jnp.ones((16,), jnp.int32), mask=ok) - 1
            plsc.store_scatter(floc_v, [idx], locc, mask=ok)
            return pos + plsc.all_reduce_population_count(ok)[0]

        cnt = lax.fori_loop(0, SCH // 16, scan, jnp.int32(0))

        def edge(e, _):
            loc = floc_v[pl.ds(e, 16)][0]
            plsc.addupdate(acc_v.at[pl.ds(loc * DEGW, 16)], one16)
            return 0

        lax.fori_loop(0, cnt, edge, 0)
        return 0

    lax.fori_loop(0, NSC, superchunk, 0)
    pltpu.sync_copy(acc_v, deg_hbm.at[pl.ds(base * DEGW, NR * DEGW)])


_sc_prep = functools.partial(
    pl.kernel,
    out_type=jax.ShapeDtypeStruct((NPAD * DEGW,), _f32),
    mesh=plsc.VectorSubcoreMesh(core_axis_name="c", subcore_axis_name="s"),
    scratch_types=[
        pltpu.VMEM((SCH,), jnp.int32),
        pltpu.VMEM((SCH + 16,), jnp.int32),
        pltpu.VMEM((NR * DEGW,), _f32),
    ],
    compiler_params=pltpu.CompilerParams(needs_layout_passes=False),
)(_prep_body)


def _agg_body(p_hbm, src_hbm, dst_hbm, s_hbm, src_v, dst_v, fsrc_v, floc_v,
              buf_v, acc_v, sem):
    c = lax.axis_index("c")
    s = lax.axis_index("s")
    w = s * NC + c
    base = w * NR
    z16 = jnp.zeros((16,), _f32)
    zi16 = jnp.zeros((16,), jnp.int32)

    def zacc(i, _):
        acc_v[pl.ds(i * 16, 16)] = z16
        return 0

    lax.fori_loop(0, NR * D // 16, zacc, 0)

    for i in range((SCH + 16) // 16):
        fsrc_v[pl.ds(i * 16, 16)] = zi16
        floc_v[pl.ds(i * 16, 16)] = zi16

    def superchunk(t, _):
        pltpu.sync_copy(src_hbm.at[pl.ds(t * SCH, SCH)], src_v)
        pltpu.sync_copy(dst_hbm.at[pl.ds(t * SCH, SCH)], dst_v)

        def scan(g, pos):
            dstv = dst_v[pl.ds(g * 16, 16)]
            loc = dstv - base
            ok = (loc >= 0) & (loc < NR)
            srcv = src_v[pl.ds(g * 16, 16)]
            locc = jnp.where(ok, loc, 0)
            idx = pos + plsc.cumsum(jnp.ones((16,), jnp.int32), mask=ok) - 1
            plsc.store_scatter(fsrc_v, [idx], srcv, mask=ok)
            plsc.store_scatter(floc_v, [idx], locc, mask=ok)
            return pos + plsc.all_reduce_population_count(ok)[0]

        cnt = lax.fori_loop(0, SCH // 16, scan, jnp.int32(0))
        nr_rounds = (cnt + GR - 1) // GR

        def rnd(r, _):
            pltpu.async_copy(
                p_hbm.at[fsrc_v.at[pl.ds(r * GR, GR)]], buf_v, sem
            ).wait()
            e_hi = jnp.minimum(GR, cnt - r * GR)

            def edge(e, _):
                loc = floc_v[pl.ds(r * GR + e, 16)][0]
                for g in range(D // 16):
                    xv = buf_v[e, pl.ds(g * 16, 16)]
                    plsc.addupdate(acc_v.at[pl.ds(loc * D + g * 16, 16)], xv)
                return 0

            lax.fori_loop(0, e_hi, edge, 0)
            return 0

        lax.fori_loop(0, nr_rounds, rnd, 0)
        return 0

    lax.fori_loop(0, NSC, superchunk, 0)
    pltpu.sync_copy(acc_v, s_hbm.at[pl.ds(base * D, NR * D)])


_sc_agg = functools.partial(
    pl.kernel,
    out_type=jax.ShapeDtypeStruct((NPAD * D,), _f32),
    mesh=plsc.VectorSubcoreMesh(core_axis_name="c", subcore_axis_name="s"),
    scratch_types=[
        pltpu.VMEM((SCH,), jnp.int32),
        pltpu.VMEM((SCH,), jnp.int32),
        pltpu.VMEM((SCH + 16,), jnp.int32),
        pltpu.VMEM((SCH + 16,), jnp.int32),
        pltpu.VMEM((GR, D), _f32),
        pltpu.VMEM((NR * D,), _f32),
        pltpu.SemaphoreType.DMA,
    ],
    compiler_params=pltpu.CompilerParams(needs_layout_passes=False),
)(_agg_body)


def _tc1_body(x_ref, deg_ref, wpre_ref, bpre_ref, wmp1_ref, out_ref):
    h = jnp.dot(x_ref[...], wpre_ref[...], preferred_element_type=_f32)
    h = jnp.maximum(h + bpre_ref[...], 0.0)
    p = jnp.dot(h, wmp1_ref[...], preferred_element_type=_f32)
    dinv = lax.rsqrt(deg_ref[...] + 1.0)
    out_ref[...] = p * dinv


def _tc2_body(s_ref, p_ref, deg_ref, b_ref, w_ref, out_ref):
    dinv = lax.rsqrt(deg_ref[...] + 1.0)
    h = jnp.maximum(dinv * (s_ref[...] + p_ref[...]) + b_ref[...], 0.0)
    out_ref[...] = jnp.dot(h, w_ref[...], preferred_element_type=_f32) * dinv


def _tc3_body(s_ref, p_ref, deg_ref, b2_ref, wpo_ref, bpo_ref, wfc_ref, bfc_ref,
              out_ref):
    dinv = lax.rsqrt(deg_ref[...] + 1.0)
    h = jnp.maximum(dinv * (s_ref[...] + p_ref[...]) + b2_ref[...], 0.0)
    h = jnp.dot(h, wpo_ref[...], preferred_element_type=_f32)
    h = jnp.maximum(h + bpo_ref[...], 0.0)
    out_ref[...] = jnp.dot(h, wfc_ref[...], preferred_element_type=_f32) + bfc_ref[...]


_BLK = 256
_GRID = NPAD // _BLK


def _row_spec():
    return pl.BlockSpec((_BLK, D), lambda i: (i, 0))


def _deg_spec():
    return pl.BlockSpec((_BLK, 1), lambda i: (i, 0))


def _w_spec():
    return pl.BlockSpec((D, D), lambda i: (0, 0))


def _b_spec():
    return pl.BlockSpec((1, D), lambda i: (0, 0))


_tc1 = pl.pallas_call(
    _tc1_body,
    grid=(_GRID,),
    in_specs=[_row_spec(), _deg_spec(), _w_spec(), _b_spec(), _w_spec()],
    out_specs=_row_spec(),
    out_shape=jax.ShapeDtypeStruct((NPAD, D), _f32),
)

_tc2 = pl.pallas_call(
    _tc2_body,
    grid=(_GRID,),
    in_specs=[_row_spec(), _row_spec(), _deg_spec(), _b_spec(), _w_spec()],
    out_specs=_row_spec(),
    out_shape=jax.ShapeDtypeStruct((NPAD, D), _f32),
)

_tc3 = pl.pallas_call(
    _tc3_body,
    grid=(_GRID,),
    in_specs=[_row_spec(), _row_spec(), _deg_spec(), _b_spec(), _w_spec(),
              _b_spec(), _w_spec(), _b_spec()],
    out_specs=_row_spec(),
    out_shape=jax.ShapeDtypeStruct((NPAD, D), _f32),
)


def kernel(x, edge_index, W_pre, b_pre, W_mp1, b_mp1, W_mp2, b_mp2, W_post,
           b_post, W_fc, b_fc):
    src = edge_index[0]
    dst = edge_index[1]
    deg = _sc_prep(dst).reshape(NPAD, DEGW)[:, :1]
    xp = jnp.pad(x, ((0, NPAD - N), (0, 0)))
    p1 = _tc1(xp, deg, W_pre, b_pre.reshape(1, D), W_mp1)
    s1 = _sc_agg(p1, src, dst).reshape(NPAD, D)
    p2 = _tc2(s1, p1, deg, b_mp1.reshape(1, D), W_mp2)
    s2 = _sc_agg(p2, src, dst).reshape(NPAD, D)
    out = _tc3(s2, p2, deg, b_mp2.reshape(1, D), W_post, b_post.reshape(1, D),
               W_fc, b_fc.reshape(1, D))
    return out[:N]


# speedup vs baseline: 5.1922x; 1.1013x over previous
import functools

import jax
import jax.numpy as jnp
from jax import lax
from jax.experimental import pallas as pl
from jax.experimental.pallas import tpu as pltpu
from jax.experimental.pallas import tpu_sc as plsc

N = 10000
E = 160000
D = 256
NPAD = 10240
NC = 2
NSUB = 16
NW = NC * NSUB
NR = NPAD // NW
SCH = 6400
NSC = E // SCH
GR = 16
DEGW = 16

_f32 = jnp.float32


def _prep_body(dst_hbm, deg_hbm, dst_v, floc_v, acc_v):
    c = lax.axis_index("c")
    s = lax.axis_index("s")
    w = s * NC + c
    base = w * NR
    z16 = jnp.zeros((16,), _f32)
    zi16 = jnp.zeros((16,), jnp.int32)
    one16 = jnp.ones((16,), _f32)

    def zacc(i, _):
        acc_v[pl.ds(i * 16, 16)] = z16
        return 0

    lax.fori_loop(0, NR * DEGW // 16, zacc, 0)

    for i in range((SCH + 16) // 16):
        floc_v[pl.ds(i * 16, 16)] = zi16

    def superchunk(t, _):
        pltpu.sync_copy(dst_hbm.at[pl.ds(t * SCH, SCH)], dst_v)

        def scan(g, pos):
            dstv = dst_v[pl.ds(g * 16, 16)]
            loc = dstv - base
            ok = (loc >= 0) & (loc < NR)
            locc = jnp.where(ok, loc, 0)
            idx = pos + plsc.cumsum(jnp.ones((16,), jnp.int32), mask=ok) - 1
            plsc.store_scatter(floc_v, [idx], locc, mask=ok)
            return pos + plsc.all_reduce_population_count(ok)[0]

        cnt = lax.fori_loop(0, SCH // 16, scan, jnp.int32(0))

        def edge(e, _):
            loc = floc_v[pl.ds(e, 16)][0]
            plsc.addupdate(acc_v.at[pl.ds(loc * DEGW, 16)], one16)
            return 0

        lax.fori_loop(0, cnt, edge, 0)
        return 0

    lax.fori_loop(0, NSC, superchunk, 0)
    pltpu.sync_copy(acc_v, deg_hbm.at[pl.ds(base * DEGW, NR * DEGW)])


_sc_prep = functools.partial(
    pl.kernel,
    out_type=jax.ShapeDtypeStruct((NPAD * DEGW,), _f32),
    mesh=plsc.VectorSubcoreMesh(core_axis_name="c", subcore_axis_name="s"),
    scratch_types=[
        pltpu.VMEM((SCH,), jnp.int32),
        pltpu.VMEM((SCH + 16,), jnp.int32),
        pltpu.VMEM((NR * DEGW,), _f32),
    ],
    compiler_params=pltpu.CompilerParams(needs_layout_passes=False),
)(_prep_body)


def _agg_body(p_hbm, src_hbm, dst_hbm, s_hbm, src_v, dst_v, fsrc_v, floc_v,
              buf_v, acc_v, sem):
    c = lax.axis_index("c")
    s = lax.axis_index("s")
    w = s * NC + c
    base = w * NR
    z16 = jnp.zeros((16,), _f32)
    zi16 = jnp.zeros((16,), jnp.int32)

    def zacc(i, _):
        acc_v[pl.ds(i * 16, 16)] = z16
        return 0

    lax.fori_loop(0, NR * D // 16, zacc, 0)

    for i in range((SCH + 16) // 16):
        fsrc_v[pl.ds(i * 16, 16)] = zi16
        floc_v[pl.ds(i * 16, 16)] = zi16

    def superchunk(t, _):
        pltpu.sync_copy(src_hbm.at[pl.ds(t * SCH, SCH)], src_v)
        pltpu.sync_copy(dst_hbm.at[pl.ds(t * SCH, SCH)], dst_v)

        def scan(g, pos):
            dstv = dst_v[pl.ds(g * 16, 16)]
            loc = dstv - base
            ok = (loc >= 0) & (loc < NR)
            srcv = src_v[pl.ds(g * 16, 16)]
            locc = jnp.where(ok, loc, 0)
            idx = pos + plsc.cumsum(jnp.ones((16,), jnp.int32), mask=ok) - 1
            plsc.store_scatter(fsrc_v, [idx], srcv, mask=ok)
            plsc.store_scatter(floc_v, [idx], locc, mask=ok)
            return pos + plsc.all_reduce_population_count(ok)[0]

        cnt = lax.fori_loop(0, SCH // 16, scan, jnp.int32(0))
        nr_rounds = (cnt + GR - 1) // GR

        def rnd(r, _):
            pltpu.async_copy(
                p_hbm.at[fsrc_v.at[pl.ds(r * GR, GR)]], buf_v, sem
            ).wait()
            e_hi = jnp.minimum(GR, cnt - r * GR)

            def edge(e, _):
                loc = floc_v[pl.ds(r * GR + e, 16)][0]
                for g in range(D // 16):
                    xv = buf_v[e, pl.ds(g * 16, 16)]
                    plsc.addupdate(acc_v.at[pl.ds(loc * D + g * 16, 16)], xv)
                return 0

            lax.fori_loop(0, e_hi, edge, 0)
            return 0

        lax.fori_loop(0, nr_rounds, rnd, 0)
        return 0

    lax.fori_loop(0, NSC, superchunk, 0)
    pltpu.sync_copy(acc_v, s_hbm.at[pl.ds(base * D, NR * D)])


_sc_agg = functools.partial(
    pl.kernel,
    out_type=jax.ShapeDtypeStruct((NPAD * D,), _f32),
    mesh=plsc.VectorSubcoreMesh(core_axis_name="c", subcore_axis_name="s"),
    scratch_types=[
        pltpu.VMEM((SCH,), jnp.int32),
        pltpu.VMEM((SCH,), jnp.int32),
        pltpu.VMEM((SCH + 16,), jnp.int32),
        pltpu.VMEM((SCH + 16,), jnp.int32),
        pltpu.VMEM((GR, D), _f32),
        pltpu.VMEM((NR * D,), _f32),
        pltpu.SemaphoreType.DMA,
    ],
    compiler_params=pltpu.CompilerParams(needs_layout_passes=False),
)(_agg_body)


def _tc1_body(x_ref, deg_ref, wpre_ref, bpre_ref, wmp1_ref, out_ref):
    h = jnp.dot(x_ref[...], wpre_ref[...], preferred_element_type=_f32)
    h = jnp.maximum(h + bpre_ref[...], 0.0)
    p = jnp.dot(h, wmp1_ref[...], preferred_element_type=_f32)
    dinv = lax.rsqrt(deg_ref[...] + 1.0)
    out_ref[...] = p * dinv


def _tc2_body(s_ref, p_ref, deg_ref, b_ref, w_ref, out_ref):
    dinv = lax.rsqrt(deg_ref[...] + 1.0)
    h = jnp.maximum(dinv * (s_ref[...] + p_ref[...]) + b_ref[...], 0.0)
    out_ref[...] = jnp.dot(h, w_ref[...], preferred_element_type=_f32) * dinv


def _tc3_body(s_ref, p_ref, deg_ref, b2_ref, wpo_ref, bpo_ref, wfc_ref, bfc_ref,
              out_ref):
    dinv = lax.rsqrt(deg_ref[...] + 1.0)
    h = jnp.maximum(dinv * (s_ref[...] + p_ref[...]) + b2_ref[...], 0.0)
    h = jnp.dot(h, wpo_ref[...], preferred_element_type=_f32)
    h = jnp.maximum(h + bpo_ref[...], 0.0)
    out_ref[...] = jnp.dot(h, wfc_ref[...], preferred_element_type=_f32) + bfc_ref[...]


_BLK = 256
_GRID = NPAD // _BLK


def _row_spec():
    return pl.BlockSpec((_BLK, D), lambda i: (i, 0))


def _deg_spec():
    return pl.BlockSpec((_BLK, 1), lambda i: (i, 0))


def _w_spec():
    return pl.BlockSpec((D, D), lambda i: (0, 0))


def _b_spec():
    return pl.BlockSpec((1, D), lambda i: (0, 0))


_tc1 = pl.pallas_call(
    _tc1_body,
    grid=(_GRID,),
    in_specs=[_row_spec(), _deg_spec(), _w_spec(), _b_spec(), _w_spec()],
    out_specs=_row_spec(),
    out_shape=jax.ShapeDtypeStruct((NPAD, D), _f32),
)

_tc2 = pl.pallas_call(
    _tc2_body,
    grid=(_GRID,),
    in_specs=[_row_spec(), _row_spec(), _deg_spec(), _b_spec(), _w_spec()],
    out_specs=_row_spec(),
    out_shape=jax.ShapeDtypeStruct((NPAD, D), _f32),
)

_tc3 = pl.pallas_call(
    _tc3_body,
    grid=(_GRID,),
    in_specs=[_row_spec(), _row_spec(), _deg_spec(), _b_spec(), _w_spec(),
              _b_spec(), _w_spec(), _b_spec()],
    out_specs=_row_spec(),
    out_shape=jax.ShapeDtypeStruct((NPAD, D), _f32),
)


def kernel(x, edge_index, W_pre, b_pre, W_mp1, b_mp1, W_mp2, b_mp2, W_post,
           b_post, W_fc, b_fc):
    src = edge_index[0]
    dst = edge_index[1]
    deg = _sc_prep(dst).reshape(NPAD, DEGW)[:, :1]
    xp = jnp.pad(x, ((0, NPAD - N), (0, 0)))
    p1 = _tc1(xp, deg, W_pre, b_pre.reshape(1, D), W_mp1)
    s1 = _sc_agg(p1, src, dst).reshape(NPAD, D)
    p2 = _tc2(s1, p1, deg, b_mp1.reshape(1, D), W_mp2)
    s2 = _sc_agg(p2, src, dst).reshape(NPAD, D)
    out = _tc3(s2, p2, deg, b_mp2.reshape(1, D), W_post, b_post.reshape(1, D),
               W_fc, b_fc.reshape(1, D))
    return out[:N]


# speedup vs baseline: 5.6168x; 1.0818x over previous
import functools

import jax
import jax.numpy as jnp
from jax import lax
from jax.experimental import pallas as pl
from jax.experimental.pallas import tpu as pltpu
from jax.experimental.pallas import tpu_sc as plsc

N = 10000
E = 160000
D = 256
NPAD = 10240
NC = 2
NSUB = 16
NW = NC * NSUB
NR = NPAD // NW
SCH = 6400
NSC = E // SCH
GR = 32
DEGW = 16

_f32 = jnp.float32


def _prep_body(dst_hbm, deg_hbm, dst_v, floc_v, acc_v):
    c = lax.axis_index("c")
    s = lax.axis_index("s")
    w = s * NC + c
    base = w * NR
    z16 = jnp.zeros((16,), _f32)
    zi16 = jnp.zeros((16,), jnp.int32)
    one16 = jnp.ones((16,), _f32)

    def zacc(i, _):
        acc_v[pl.ds(i * 16, 16)] = z16
        return 0

    lax.fori_loop(0, NR * DEGW // 16, zacc, 0)

    for i in range((SCH + 16) // 16):
        floc_v[pl.ds(i * 16, 16)] = zi16

    def superchunk(t, _):
        pltpu.sync_copy(dst_hbm.at[pl.ds(t * SCH, SCH)], dst_v)

        def scan(g, pos):
            dstv = dst_v[pl.ds(g * 16, 16)]
            loc = dstv - base
            ok = (loc >= 0) & (loc < NR)
            locc = jnp.where(ok, loc, 0)
            idx = pos + plsc.cumsum(jnp.ones((16,), jnp.int32), mask=ok) - 1
            plsc.store_scatter(floc_v, [idx], locc, mask=ok)
            return pos + plsc.all_reduce_population_count(ok)[0]

        cnt = lax.fori_loop(0, SCH // 16, scan, jnp.int32(0))

        def edge(e, _):
            loc = floc_v[pl.ds(e, 16)][0]
            plsc.addupdate(acc_v.at[pl.ds(loc * DEGW, 16)], one16)
            return 0

        lax.fori_loop(0, cnt, edge, 0)
        return 0

    lax.fori_loop(0, NSC, superchunk, 0)
    pltpu.sync_copy(acc_v, deg_hbm.at[pl.ds(base * DEGW, NR * DEGW)])


_sc_prep = functools.partial(
    pl.kernel,
    out_type=jax.ShapeDtypeStruct((NPAD * DEGW,), _f32),
    mesh=plsc.VectorSubcoreMesh(core_axis_name="c", subcore_axis_name="s"),
    scratch_types=[
        pltpu.VMEM((SCH,), jnp.int32),
        pltpu.VMEM((SCH + 16,), jnp.int32),
        pltpu.VMEM((NR * DEGW,), _f32),
    ],
    compiler_params=pltpu.CompilerParams(needs_layout_passes=False),
)(_prep_body)


def _agg_body(p_hbm, src_hbm, dst_hbm, s_hbm, src_v, dst_v, fsrc_v, floc_v,
              buf_v, acc_v, sem):
    c = lax.axis_index("c")
    s = lax.axis_index("s")
    w = s * NC + c
    base = w * NR
    z16 = jnp.zeros((16,), _f32)
    zi16 = jnp.zeros((16,), jnp.int32)

    def zacc(i, _):
        acc_v[pl.ds(i * 16, 16)] = z16
        return 0

    lax.fori_loop(0, NR * D // 16, zacc, 0)

    for i in range((SCH + 16) // 16):
        fsrc_v[pl.ds(i * 16, 16)] = zi16
        floc_v[pl.ds(i * 16, 16)] = zi16

    def superchunk(t, _):
        pltpu.sync_copy(src_hbm.at[pl.ds(t * SCH, SCH)], src_v)
        pltpu.sync_copy(dst_hbm.at[pl.ds(t * SCH, SCH)], dst_v)

        def scan(g, pos):
            dstv = dst_v[pl.ds(g * 16, 16)]
            loc = dstv - base
            ok = (loc >= 0) & (loc < NR)
            srcv = src_v[pl.ds(g * 16, 16)]
            locc = jnp.where(ok, loc, 0)
            idx = pos + plsc.cumsum(jnp.ones((16,), jnp.int32), mask=ok) - 1
            plsc.store_scatter(fsrc_v, [idx], srcv, mask=ok)
            plsc.store_scatter(floc_v, [idx], locc, mask=ok)
            return pos + plsc.all_reduce_population_count(ok)[0]

        cnt = lax.fori_loop(0, SCH // 16, scan, jnp.int32(0))
        nr_rounds = (cnt + GR - 1) // GR

        def rnd(r, _):
            pltpu.async_copy(
                p_hbm.at[fsrc_v.at[pl.ds(r * GR, GR)]], buf_v, sem
            ).wait()
            e_hi = jnp.minimum(GR, cnt - r * GR)

            def edge(e, _):
                loc = floc_v[pl.ds(r * GR + e, 16)][0]
                for g in range(D // 16):
                    xv = buf_v[e, pl.ds(g * 16, 16)]
                    plsc.addupdate(acc_v.at[pl.ds(loc * D + g * 16, 16)], xv)
                return 0

            lax.fori_loop(0, e_hi, edge, 0)
            return 0

        lax.fori_loop(0, nr_rounds, rnd, 0)
        return 0

    lax.fori_loop(0, NSC, superchunk, 0)
    pltpu.sync_copy(acc_v, s_hbm.at[pl.ds(base * D, NR * D)])


_sc_agg = functools.partial(
    pl.kernel,
    out_type=jax.ShapeDtypeStruct((NPAD * D,), _f32),
    mesh=plsc.VectorSubcoreMesh(core_axis_name="c", subcore_axis_name="s"),
    scratch_types=[
        pltpu.VMEM((SCH,), jnp.int32),
        pltpu.VMEM((SCH,), jnp.int32),
        pltpu.VMEM((SCH + 16,), jnp.int32),
        pltpu.VMEM((SCH + 16,), jnp.int32),
        pltpu.VMEM((GR, D), _f32),
        pltpu.VMEM((NR * D,), _f32),
        pltpu.SemaphoreType.DMA,
    ],
    compiler_params=pltpu.CompilerParams(needs_layout_passes=False),
)(_agg_body)


def _tc1_body(x_ref, deg_ref, wpre_ref, bpre_ref, wmp1_ref, out_ref):
    h = jnp.dot(x_ref[...], wpre_ref[...], preferred_element_type=_f32)
    h = jnp.maximum(h + bpre_ref[...], 0.0)
    p = jnp.dot(h, wmp1_ref[...], preferred_element_type=_f32)
    dinv = lax.rsqrt(deg_ref[...] + 1.0)
    out_ref[...] = p * dinv


def _tc2_body(s_ref, p_ref, deg_ref, b_ref, w_ref, out_ref):
    dinv = lax.rsqrt(deg_ref[...] + 1.0)
    h = jnp.maximum(dinv * (s_ref[...] + p_ref[...]) + b_ref[...], 0.0)
    out_ref[...] = jnp.dot(h, w_ref[...], preferred_element_type=_f32) * dinv


def _tc3_body(s_ref, p_ref, deg_ref, b2_ref, wpo_ref, bpo_ref, wfc_ref, bfc_ref,
              out_ref):
    dinv = lax.rsqrt(deg_ref[...] + 1.0)
    h = jnp.maximum(dinv * (s_ref[...] + p_ref[...]) + b2_ref[...], 0.0)
    h = jnp.dot(h, wpo_ref[...], preferred_element_type=_f32)
    h = jnp.maximum(h + bpo_ref[...], 0.0)
    out_ref[...] = jnp.dot(h, wfc_ref[...], preferred_element_type=_f32) + bfc_ref[...]


_BLK = 256
_GRID = NPAD // _BLK


def _row_spec():
    return pl.BlockSpec((_BLK, D), lambda i: (i, 0))


def _deg_spec():
    return pl.BlockSpec((_BLK, 1), lambda i: (i, 0))


def _w_spec():
    return pl.BlockSpec((D, D), lambda i: (0, 0))


def _b_spec():
    return pl.BlockSpec((1, D), lambda i: (0, 0))


_tc1 = pl.pallas_call(
    _tc1_body,
    grid=(_GRID,),
    in_specs=[_row_spec(), _deg_spec(), _w_spec(), _b_spec(), _w_spec()],
    out_specs=_row_spec(),
    out_shape=jax.ShapeDtypeStruct((NPAD, D), _f32),
)

_tc2 = pl.pallas_call(
    _tc2_body,
    grid=(_GRID,),
    in_specs=[_row_spec(), _row_spec(), _deg_spec(), _b_spec(), _w_spec()],
    out_specs=_row_spec(),
    out_shape=jax.ShapeDtypeStruct((NPAD, D), _f32),
)

_tc3 = pl.pallas_call(
    _tc3_body,
    grid=(_GRID,),
    in_specs=[_row_spec(), _row_spec(), _deg_spec(), _b_spec(), _w_spec(),
              _b_spec(), _w_spec(), _b_spec()],
    out_specs=_row_spec(),
    out_shape=jax.ShapeDtypeStruct((NPAD, D), _f32),
)


def kernel(x, edge_index, W_pre, b_pre, W_mp1, b_mp1, W_mp2, b_mp2, W_post,
           b_post, W_fc, b_fc):
    src = edge_index[0]
    dst = edge_index[1]
    deg = _sc_prep(dst).reshape(NPAD, DEGW)[:, :1]
    xp = jnp.pad(x, ((0, NPAD - N), (0, 0)))
    p1 = _tc1(xp, deg, W_pre, b_pre.reshape(1, D), W_mp1)
    s1 = _sc_agg(p1, src, dst).reshape(NPAD, D)
    p2 = _tc2(s1, p1, deg, b_mp1.reshape(1, D), W_mp2)
    s2 = _sc_agg(p2, src, dst).reshape(NPAD, D)
    out = _tc3(s2, p2, deg, b_mp2.reshape(1, D), W_post, b_post.reshape(1, D),
               W_fc, b_fc.reshape(1, D))
    return out[:N]


# speedup vs baseline: 5.6290x; 1.0022x over previous
import functools

import jax
import jax.numpy as jnp
from jax import lax
from jax.experimental import pallas as pl
from jax.experimental.pallas import tpu as pltpu
from jax.experimental.pallas import tpu_sc as plsc

N = 10000
E = 160000
D = 256
NPAD = 10240
NC = 2
NSUB = 16
NW = NC * NSUB
NR = NPAD // NW
SCH = 8000
NSC = E // SCH
GR = 32
DEGW = 16

_f32 = jnp.float32


def _prep_body(dst_hbm, deg_hbm, dst_v, floc_v, acc_v):
    c = lax.axis_index("c")
    s = lax.axis_index("s")
    w = s * NC + c
    base = w * NR
    z16 = jnp.zeros((16,), _f32)
    zi16 = jnp.zeros((16,), jnp.int32)
    one16 = jnp.ones((16,), _f32)

    def zacc(i, _):
        acc_v[pl.ds(i * 16, 16)] = z16
        return 0

    lax.fori_loop(0, NR * DEGW // 16, zacc, 0)

    for i in range((SCH + 16) // 16):
        floc_v[pl.ds(i * 16, 16)] = zi16

    def superchunk(t, _):
        pltpu.sync_copy(dst_hbm.at[pl.ds(t * SCH, SCH)], dst_v)

        def scan(g, pos):
            dstv = dst_v[pl.ds(g * 16, 16)]
            loc = dstv - base
            ok = (loc >= 0) & (loc < NR)
            locc = jnp.where(ok, loc, 0)
            idx = pos + plsc.cumsum(jnp.ones((16,), jnp.int32), mask=ok) - 1
            plsc.store_scatter(floc_v, [idx], locc, mask=ok)
            return pos + plsc.all_reduce_population_count(ok)[0]

        cnt = lax.fori_loop(0, SCH // 16, scan, jnp.int32(0))

        def edge(e, _):
            loc = floc_v[pl.ds(e, 16)][0]
            plsc.addupdate(acc_v.at[pl.ds(loc * DEGW, 16)], one16)
            return 0

        lax.fori_loop(0, cnt, edge, 0)
        return 0

    lax.fori_loop(0, NSC, superchunk, 0)
    pltpu.sync_copy(acc_v, deg_hbm.at[pl.ds(base * DEGW, NR * DEGW)])


_sc_prep = functools.partial(
    pl.kernel,
    out_type=jax.ShapeDtypeStruct((NPAD * DEGW,), _f32),
    mesh=plsc.VectorSubcoreMesh(core_axis_name="c", subcore_axis_name="s"),
    scratch_types=[
        pltpu.VMEM((SCH,), jnp.int32),
        pltpu.VMEM((SCH + 16,), jnp.int32),
        pltpu.VMEM((NR * DEGW,), _f32),
    ],
    compiler_params=pltpu.CompilerParams(needs_layout_passes=False),
)(_prep_body)


def _agg_body(p_hbm, src_hbm, dst_hbm, s_hbm, src_v, dst_v, fsrc_v, floc_v,
              buf_v, acc_v, sem):
    c = lax.axis_index("c")
    s = lax.axis_index("s")
    w = s * NC + c
    base = w * NR
    z16 = jnp.zeros((16,), _f32)
    zi16 = jnp.zeros((16,), jnp.int32)

    def zacc(i, _):
        acc_v[pl.ds(i * 16, 16)] = z16
        return 0

    lax.fori_loop(0, NR * D // 16, zacc, 0)

    for i in range((SCH + 16) // 16):
        fsrc_v[pl.ds(i * 16, 16)] = zi16
        floc_v[pl.ds(i * 16, 16)] = zi16

    def superchunk(t, _):
        pltpu.sync_copy(src_hbm.at[pl.ds(t * SCH, SCH)], src_v)
        pltpu.sync_copy(dst_hbm.at[pl.ds(t * SCH, SCH)], dst_v)

        def scan(g, pos):
            dstv = dst_v[pl.ds(g * 16, 16)]
            loc = dstv - base
            ok = (loc >= 0) & (loc < NR)
            srcv = src_v[pl.ds(g * 16, 16)]
            locc = jnp.where(ok, loc, 0)
            idx = pos + plsc.cumsum(jnp.ones((16,), jnp.int32), mask=ok) - 1
            plsc.store_scatter(fsrc_v, [idx], srcv, mask=ok)
            plsc.store_scatter(floc_v, [idx], locc, mask=ok)
            return pos + plsc.all_reduce_population_count(ok)[0]

        cnt = lax.fori_loop(0, SCH // 16, scan, jnp.int32(0))
        nr_rounds = (cnt + GR - 1) // GR

        def rnd(r, _):
            pltpu.async_copy(
                p_hbm.at[fsrc_v.at[pl.ds(r * GR, GR)]], buf_v, sem
            ).wait()
            e_hi = jnp.minimum(GR, cnt - r * GR)

            def edge(e, _):
                loc = floc_v[pl.ds(r * GR + e, 16)][0]
                for g in range(D // 16):
                    xv = buf_v[e, pl.ds(g * 16, 16)]
                    plsc.addupdate(acc_v.at[pl.ds(loc * D + g * 16, 16)], xv)
                return 0

            lax.fori_loop(0, e_hi, edge, 0)
            return 0

        lax.fori_loop(0, nr_rounds, rnd, 0)
        return 0

    lax.fori_loop(0, NSC, superchunk, 0)
    pltpu.sync_copy(acc_v, s_hbm.at[pl.ds(base * D, NR * D)])


_sc_agg = functools.partial(
    pl.kernel,
    out_type=jax.ShapeDtypeStruct((NPAD * D,), _f32),
    mesh=plsc.VectorSubcoreMesh(core_axis_name="c", subcore_axis_name="s"),
    scratch_types=[
        pltpu.VMEM((SCH,), jnp.int32),
        pltpu.VMEM((SCH,), jnp.int32),
        pltpu.VMEM((SCH + 16,), jnp.int32),
        pltpu.VMEM((SCH + 16,), jnp.int32),
        pltpu.VMEM((GR, D), _f32),
        pltpu.VMEM((NR * D,), _f32),
        pltpu.SemaphoreType.DMA,
    ],
    compiler_params=pltpu.CompilerParams(needs_layout_passes=False),
)(_agg_body)


def _tc1_body(x_ref, deg_ref, wpre_ref, bpre_ref, wmp1_ref, out_ref):
    h = jnp.dot(x_ref[...], wpre_ref[...], preferred_element_type=_f32)
    h = jnp.maximum(h + bpre_ref[...], 0.0)
    p = jnp.dot(h, wmp1_ref[...], preferred_element_type=_f32)
    dinv = lax.rsqrt(deg_ref[...] + 1.0)
    out_ref[...] = p * dinv


def _tc2_body(s_ref, p_ref, deg_ref, b_ref, w_ref, out_ref):
    dinv = lax.rsqrt(deg_ref[...] + 1.0)
    h = jnp.maximum(dinv * (s_ref[...] + p_ref[...]) + b_ref[...], 0.0)
    out_ref[...] = jnp.dot(h, w_ref[...], preferred_element_type=_f32) * dinv


def _tc3_body(s_ref, p_ref, deg_ref, b2_ref, wpo_ref, bpo_ref, wfc_ref, bfc_ref,
              out_ref):
    dinv = lax.rsqrt(deg_ref[...] + 1.0)
    h = jnp.maximum(dinv * (s_ref[...] + p_ref[...]) + b2_ref[...], 0.0)
    h = jnp.dot(h, wpo_ref[...], preferred_element_type=_f32)
    h = jnp.maximum(h + bpo_ref[...], 0.0)
    out_ref[...] = jnp.dot(h, wfc_ref[...], preferred_element_type=_f32) + bfc_ref[...]


_BLK = 256
_GRID = NPAD // _BLK


def _row_spec():
    return pl.BlockSpec((_BLK, D), lambda i: (i, 0))


def _deg_spec():
    return pl.BlockSpec((_BLK, 1), lambda i: (i, 0))


def _w_spec():
    return pl.BlockSpec((D, D), lambda i: (0, 0))


def _b_spec():
    return pl.BlockSpec((1, D), lambda i: (0, 0))


_tc1 = pl.pallas_call(
    _tc1_body,
    grid=(_GRID,),
    in_specs=[_row_spec(), _deg_spec(), _w_spec(), _b_spec(), _w_spec()],
    out_specs=_row_spec(),
    out_shape=jax.ShapeDtypeStruct((NPAD, D), _f32),
)

_tc2 = pl.pallas_call(
    _tc2_body,
    grid=(_GRID,),
    in_specs=[_row_spec(), _row_spec(), _deg_spec(), _b_spec(), _w_spec()],
    out_specs=_row_spec(),
    out_shape=jax.ShapeDtypeStruct((NPAD, D), _f32),
)

_tc3 = pl.pallas_call(
    _tc3_body,
    grid=(_GRID,),
    in_specs=[_row_spec(), _row_spec(), _deg_spec(), _b_spec(), _w_spec(),
              _b_spec(), _w_spec(), _b_spec()],
    out_specs=_row_spec(),
    out_shape=jax.ShapeDtypeStruct((NPAD, D), _f32),
)


def kernel(x, edge_index, W_pre, b_pre, W_mp1, b_mp1, W_mp2, b_mp2, W_post,
           b_post, W_fc, b_fc):
    src = edge_index[0]
    dst = edge_index[1]
    deg = _sc_prep(dst).reshape(NPAD, DEGW)[:, :1]
    xp = jnp.pad(x, ((0, NPAD - N), (0, 0)))
    p1 = _tc1(xp, deg, W_pre, b_pre.reshape(1, D), W_mp1)
    s1 = _sc_agg(p1, src, dst).reshape(NPAD, D)
    p2 = _tc2(s1, p1, deg, b_mp1.reshape(1, D), W_mp2)
    s2 = _sc_agg(p2, src, dst).reshape(NPAD, D)
    out = _tc3(s2, p2, deg, b_mp2.reshape(1, D), W_post, b_post.reshape(1, D),
               W_fc, b_fc.reshape(1, D))
    return out[:N]
